# baseline (speedup 1.0000x reference)
import functools

import jax
import jax.numpy as jnp
from jax import lax
from jax.experimental import pallas as pl
from jax.experimental.pallas import tpu as pltpu
from jax.experimental.pallas import tpu_sc as plsc

N = 10000
E = 320000
R = 8
D_IN = 128
H = 64

NP = 10240
TILE = 640
GRID = NP // TILE

NCORE = 2
NSUB = 16
NWORK = NCORE * NSUB
CHUNK = 128
K = 80
EP = NWORK * K * CHUNK
IDXROWS = EP // CHUNK
ROWS_PER_SUB = NP // NSUB
NBUF = 4


def _layer0_body(h_ref, wfc_ref, wrel_ref, wloop_ref, proj_ref, loop_ref):
    x = lax.dot_general(h_ref[...], wfc_ref[...], (((1,), (1,)), ((), ())),
                        preferred_element_type=jnp.float32)
    loop_ref[...] = jnp.dot(x, wloop_ref[...], preferred_element_type=jnp.float32)
    for r in range(R):
        proj_ref[r] = jnp.dot(x, wrel_ref[r],
                              preferred_element_type=jnp.float32)


def _layer_mid_body(agg_ref, loopp_ref, b_ref, wrel_ref, wloop_ref,
                    proj_ref, loop_ref):
    x = agg_ref[0] + agg_ref[1] + loopp_ref[...] + b_ref[...]
    x = jnp.maximum(x, 0.0)
    loop_ref[...] = jnp.dot(x, wloop_ref[...], preferred_element_type=jnp.float32)
    for r in range(R):
        proj_ref[r] = jnp.dot(x, wrel_ref[r],
                              preferred_element_type=jnp.float32)


def _final_body(agg_ref, loopp_ref, b_ref, wd_ref, bd_ref, out_ref, acc_ref):
    i = pl.program_id(0)
    x = agg_ref[0] + agg_ref[1] + loopp_ref[...] + b_ref[...]
    x = jnp.maximum(x, 0.0)
    rows = lax.broadcasted_iota(jnp.int32, (TILE, H), 0) + i * TILE
    x = jnp.where(rows < N, x, 0.0)
    part = jnp.sum(x * wd_ref[...])

    @pl.when(i == 0)
    def _():
        acc_ref[0] = 0.0

    acc_ref[0] += part

    @pl.when(i == GRID - 1)
    def _():
        out_ref[0, 0, 0] = acc_ref[0] + bd_ref[0, 0]


def _tc_layer0(h_p, W_fc, W_rel, W_loop):
    return pl.pallas_call(
        _layer0_body,
        grid=(GRID,),
        in_specs=[
            pl.BlockSpec((TILE, D_IN), lambda i: (i, 0)),
            pl.BlockSpec((H, D_IN), lambda i: (0, 0)),
            pl.BlockSpec((R, H, H), lambda i: (0, 0, 0)),
            pl.BlockSpec((H, H), lambda i: (0, 0)),
        ],
        out_specs=[
            pl.BlockSpec((R, TILE, H), lambda i: (0, i, 0)),
            pl.BlockSpec((TILE, H), lambda i: (i, 0)),
        ],
        out_shape=[
            jax.ShapeDtypeStruct((R, NP, H), jnp.float32),
            jax.ShapeDtypeStruct((NP, H), jnp.float32),
        ],
    )(h_p, W_fc, W_rel, W_loop)


def _tc_layer_mid(agg, loop_prev, b_prev, W_rel, W_loop):
    return pl.pallas_call(
        _layer_mid_body,
        grid=(GRID,),
        in_specs=[
            pl.BlockSpec((2, TILE, H), lambda i: (0, i, 0)),
            pl.BlockSpec((TILE, H), lambda i: (i, 0)),
            pl.BlockSpec((1, H), lambda i: (0, 0)),
            pl.BlockSpec((R, H, H), lambda i: (0, 0, 0)),
            pl.BlockSpec((H, H), lambda i: (0, 0)),
        ],
        out_specs=[
            pl.BlockSpec((R, TILE, H), lambda i: (0, i, 0)),
            pl.BlockSpec((TILE, H), lambda i: (i, 0)),
        ],
        out_shape=[
            jax.ShapeDtypeStruct((R, NP, H), jnp.float32),
            jax.ShapeDtypeStruct((NP, H), jnp.float32),
        ],
    )(agg, loop_prev, b_prev.reshape(1, H), W_rel, W_loop)


def _tc_final(agg, loop_prev, b_prev, W_dense, b_dense):
    return pl.pallas_call(
        _final_body,
        grid=(GRID,),
        in_specs=[
            pl.BlockSpec((2, TILE, H), lambda i: (0, i, 0)),
            pl.BlockSpec((TILE, H), lambda i: (i, 0)),
            pl.BlockSpec((1, H), lambda i: (0, 0)),
            pl.BlockSpec((1, H), lambda i: (0, 0)),
            pl.BlockSpec(memory_space=pltpu.SMEM),
        ],
        out_specs=pl.BlockSpec(memory_space=pltpu.SMEM),
        out_shape=jax.ShapeDtypeStruct((1, 1, 1), jnp.float32),
        scratch_shapes=[pltpu.SMEM((1,), jnp.float32)],
    )(agg, loop_prev, b_prev.reshape(1, H), W_dense, b_dense.reshape(1, 1))


def _sc_body(proj_hbm, fidx_hbm, dst_hbm, out_hbm,
             idx_v, dsti_v, fbuf, agg_sh, gsem, ssem):
    c = lax.axis_index("c")
    s = lax.axis_index("s")
    wid = c * NSUB + s
    row0 = wid * K

    pltpu.sync_copy(fidx_hbm.at[pl.ds(row0, K)], idx_v)
    pltpu.sync_copy(dst_hbm.at[pl.ds(row0, K)], dsti_v)

    zeros16 = jnp.zeros((16,), jnp.float32)

    def _zero_row(i, carry):
        for k in range(H // 16):
            fbuf[0, i, pl.ds(k * 16, 16)] = zeros16
        return carry

    lax.fori_loop(0, CHUNK, _zero_row, 0)
    for k in range(ROWS_PER_SUB // CHUNK):
        pltpu.sync_copy(fbuf.at[0],
                        agg_sh.at[pl.ds(s * ROWS_PER_SUB + k * CHUNK, CHUNK)])
    plsc.subcore_barrier()

    for b in range(NBUF):
        pltpu.async_copy(proj_hbm.at[idx_v.at[b]], fbuf.at[b], gsem.at[b])

    def _round(t, carry):
        for b in range(NBUF):
            j = t * NBUF + b
            pltpu.make_async_copy(proj_hbm.at[idx_v.at[j]], fbuf.at[b],
                                  gsem.at[b]).wait()
            pltpu.async_copy(fbuf.at[b], agg_sh.at[dsti_v.at[j]], ssem.at[b],
                             add=True)

            @pl.when(t < K // NBUF - 1)
            def _():
                pltpu.make_async_copy(fbuf.at[b], agg_sh.at[dsti_v.at[j]],
                                      ssem.at[b]).wait()
                pltpu.async_copy(proj_hbm.at[idx_v.at[(t + 1) * NBUF + b]],
                                 fbuf.at[b], gsem.at[b])
        return carry

    lax.fori_loop(0, K // NBUF, _round, 0)
    for b in range(NBUF):
        pltpu.make_async_copy(fbuf.at[b], agg_sh.at[dsti_v.at[K - NBUF + b]],
                              ssem.at[b]).wait()
    plsc.subcore_barrier()

    pltpu.sync_copy(agg_sh.at[pl.ds(s * ROWS_PER_SUB, ROWS_PER_SUB)],
                    out_hbm.at[c, pl.ds(s * ROWS_PER_SUB, ROWS_PER_SUB)])


@functools.lru_cache(maxsize=None)
def _make_sc_gather_scatter():
    return pl.kernel(
        _sc_body,
        out_type=jax.ShapeDtypeStruct((NCORE, NP, H), jnp.float32),
        mesh=plsc.VectorSubcoreMesh(core_axis_name="c", subcore_axis_name="s"),
        scratch_types=[
            pltpu.VMEM((K, CHUNK), jnp.int32),
            pltpu.VMEM((K, CHUNK), jnp.int32),
            pltpu.VMEM((NBUF, CHUNK, H), jnp.float32),
            pltpu.VMEM_SHARED((NP, H), jnp.float32),
            pltpu.SemaphoreType.DMA((NBUF,)),
            pltpu.SemaphoreType.DMA((NBUF,)),
        ],
        compiler_params=pltpu.CompilerParams(use_tc_tiling_on_sc=False),
    )


def _sc_gather_scatter(proj_flat, fidx_p, dst_p):
    return _make_sc_gather_scatter()(proj_flat, fidx_p, dst_p)


def kernel(h, edge_index, etypes, W_fc, W_attn, W_rel0, W_loop0, b0,
           W_rel1, W_loop1, b1, W_rel2, W_loop2, b2, W_dense, b_dense):
    src = edge_index[0]
    dst = edge_index[1]

    fidx = etypes * NP + src
    pad = EP - E
    fidx_p = jnp.concatenate([fidx, jnp.zeros((pad,), jnp.int32)]).reshape(IDXROWS, CHUNK)
    dst_p = jnp.concatenate([dst, jnp.full((pad,), N, jnp.int32)]).reshape(IDXROWS, CHUNK)
    h_p = jnp.pad(h, ((0, NP - N), (0, 0)))

    proj0, loop0 = _tc_layer0(h_p, W_fc, W_rel0, W_loop0)
    agg0 = _sc_gather_scatter(proj0.reshape(R * NP, H), fidx_p, dst_p)
    proj1, loop1 = _tc_layer_mid(agg0, loop0, b0, W_rel1, W_loop1)
    agg1 = _sc_gather_scatter(proj1.reshape(R * NP, H), fidx_p, dst_p)
    proj2, loop2 = _tc_layer_mid(agg1, loop1, b1, W_rel2, W_loop2)
    agg2 = _sc_gather_scatter(proj2.reshape(R * NP, H), fidx_p, dst_p)
    return _tc_final(agg2, loop2, b2, W_dense, b_dense)


# speedup vs baseline: 1.1643x; 1.1643x over previous
import functools

import jax
import jax.numpy as jnp
from jax import lax
from jax.experimental import pallas as pl
from jax.experimental.pallas import tpu as pltpu
from jax.experimental.pallas import tpu_sc as plsc

N = 10000
E = 320000
R = 8
D_IN = 128
H = 64

NP = 10240
TILE = 640
GRID = NP // TILE

NCORE = 2
NSUB = 16
NWORK = NCORE * NSUB
CHUNK = 128
K = 80
EP = NWORK * K * CHUNK
IDXROWS = EP // CHUNK
ROWS_PER_SUB = NP // NSUB
NBUF = 4


def _layer0_body(h_ref, wfc_ref, wrel_ref, wloop_ref, proj_ref, loop_ref):
    x = lax.dot_general(h_ref[...], wfc_ref[...], (((1,), (1,)), ((), ())),
                        preferred_element_type=jnp.float32)
    loop_ref[...] = jnp.dot(x, wloop_ref[...], preferred_element_type=jnp.float32)
    for r in range(R):
        proj_ref[r] = jnp.dot(x, wrel_ref[r],
                              preferred_element_type=jnp.float32).astype(jnp.bfloat16)


def _layer_mid_body(agg_ref, loopp_ref, b_ref, wrel_ref, wloop_ref,
                    proj_ref, loop_ref):
    x = agg_ref[0] + agg_ref[1] + loopp_ref[...] + b_ref[...]
    x = jnp.maximum(x, 0.0)
    loop_ref[...] = jnp.dot(x, wloop_ref[...], preferred_element_type=jnp.float32)
    for r in range(R):
        proj_ref[r] = jnp.dot(x, wrel_ref[r],
                              preferred_element_type=jnp.float32).astype(jnp.bfloat16)


def _final_body(agg_ref, loopp_ref, b_ref, wd_ref, bd_ref, out_ref, acc_ref):
    i = pl.program_id(0)
    x = agg_ref[0] + agg_ref[1] + loopp_ref[...] + b_ref[...]
    x = jnp.maximum(x, 0.0)
    rows = lax.broadcasted_iota(jnp.int32, (TILE, H), 0) + i * TILE
    x = jnp.where(rows < N, x, 0.0)
    part = jnp.sum(x * wd_ref[...])

    @pl.when(i == 0)
    def _():
        acc_ref[0] = 0.0

    acc_ref[0] += part

    @pl.when(i == GRID - 1)
    def _():
        out_ref[0, 0, 0] = acc_ref[0] + bd_ref[0, 0]


def _tc_layer0(h_p, W_fc, W_rel, W_loop):
    return pl.pallas_call(
        _layer0_body,
        grid=(GRID,),
        in_specs=[
            pl.BlockSpec((TILE, D_IN), lambda i: (i, 0)),
            pl.BlockSpec((H, D_IN), lambda i: (0, 0)),
            pl.BlockSpec((R, H, H), lambda i: (0, 0, 0)),
            pl.BlockSpec((H, H), lambda i: (0, 0)),
        ],
        out_specs=[
            pl.BlockSpec((R, TILE, H), lambda i: (0, i, 0)),
            pl.BlockSpec((TILE, H), lambda i: (i, 0)),
        ],
        out_shape=[
            jax.ShapeDtypeStruct((R, NP, H), jnp.bfloat16),
            jax.ShapeDtypeStruct((NP, H), jnp.float32),
        ],
    )(h_p, W_fc, W_rel, W_loop)


def _tc_layer_mid(agg, loop_prev, b_prev, W_rel, W_loop):
    return pl.pallas_call(
        _layer_mid_body,
        grid=(GRID,),
        in_specs=[
            pl.BlockSpec((2, TILE, H), lambda i: (0, i, 0)),
            pl.BlockSpec((TILE, H), lambda i: (i, 0)),
            pl.BlockSpec((1, H), lambda i: (0, 0)),
            pl.BlockSpec((R, H, H), lambda i: (0, 0, 0)),
            pl.BlockSpec((H, H), lambda i: (0, 0)),
        ],
        out_specs=[
            pl.BlockSpec((R, TILE, H), lambda i: (0, i, 0)),
            pl.BlockSpec((TILE, H), lambda i: (i, 0)),
        ],
        out_shape=[
            jax.ShapeDtypeStruct((R, NP, H), jnp.bfloat16),
            jax.ShapeDtypeStruct((NP, H), jnp.float32),
        ],
    )(agg, loop_prev, b_prev.reshape(1, H), W_rel, W_loop)


def _tc_final(agg, loop_prev, b_prev, W_dense, b_dense):
    return pl.pallas_call(
        _final_body,
        grid=(GRID,),
        in_specs=[
            pl.BlockSpec((2, TILE, H), lambda i: (0, i, 0)),
            pl.BlockSpec((TILE, H), lambda i: (i, 0)),
            pl.BlockSpec((1, H), lambda i: (0, 0)),
            pl.BlockSpec((1, H), lambda i: (0, 0)),
            pl.BlockSpec(memory_space=pltpu.SMEM),
        ],
        out_specs=pl.BlockSpec(memory_space=pltpu.SMEM),
        out_shape=jax.ShapeDtypeStruct((1, 1, 1), jnp.float32),
        scratch_shapes=[pltpu.SMEM((1,), jnp.float32)],
    )(agg, loop_prev, b_prev.reshape(1, H), W_dense, b_dense.reshape(1, 1))


def _sc_body(proj_hbm, fidx_hbm, dst_hbm, out_hbm,
             idx_v, dsti_v, rows_bf, fbuf, agg_sh, gsem, ssem):
    c = lax.axis_index("c")
    s = lax.axis_index("s")
    wid = s * NCORE + c
    row0 = wid * K

    pltpu.sync_copy(fidx_hbm.at[pl.ds(row0, K)], idx_v)
    pltpu.sync_copy(dst_hbm.at[pl.ds(row0, K)], dsti_v)

    zeros16 = jnp.zeros((16,), jnp.float32)

    def _zero_row(i, carry):
        for k in range(H // 16):
            fbuf[0, i, pl.ds(k * 16, 16)] = zeros16
        return carry

    lax.fori_loop(0, CHUNK, _zero_row, 0)
    for k in range(ROWS_PER_SUB // CHUNK):
        pltpu.sync_copy(fbuf.at[0],
                        agg_sh.at[pl.ds(s * ROWS_PER_SUB + k * CHUNK, CHUNK)])
    plsc.subcore_barrier()

    def _widen_chunk(b, carry):
        def _row(i, carry2):
            for k in range(H // 32):
                v = rows_bf[b, i, pl.ds(k * 32, 32)]
                fbuf[b, i, pl.ds(k * 32, 32)] = v.astype(jnp.float32)
            return carry2

        return lax.fori_loop(0, CHUNK, _row, carry)

    for b in range(NBUF):
        pltpu.async_copy(proj_hbm.at[idx_v.at[b]], rows_bf.at[b], gsem.at[b])

    def _round(t, carry):
        for b in range(NBUF):
            j = t * NBUF + b
            pltpu.make_async_copy(proj_hbm.at[idx_v.at[j]], rows_bf.at[b],
                                  gsem.at[b]).wait()

            @pl.when(t > 0)
            def _():
                pltpu.make_async_copy(fbuf.at[b], agg_sh.at[dsti_v.at[j]],
                                      ssem.at[b]).wait()

            _widen_chunk(b, 0)
            pltpu.async_copy(fbuf.at[b], agg_sh.at[dsti_v.at[j]], ssem.at[b],
                             add=True)

            @pl.when(t < K // NBUF - 1)
            def _():
                pltpu.async_copy(proj_hbm.at[idx_v.at[(t + 1) * NBUF + b]],
                                 rows_bf.at[b], gsem.at[b])
        return carry

    lax.fori_loop(0, K // NBUF, _round, 0)
    for b in range(NBUF):
        pltpu.make_async_copy(fbuf.at[b], agg_sh.at[dsti_v.at[0]],
                              ssem.at[b]).wait()
    plsc.subcore_barrier()

    pltpu.sync_copy(agg_sh.at[pl.ds(s * ROWS_PER_SUB, ROWS_PER_SUB)],
                    out_hbm.at[c, pl.ds(s * ROWS_PER_SUB, ROWS_PER_SUB)])


@functools.lru_cache(maxsize=None)
def _make_sc_gather_scatter():
    return pl.kernel(
        _sc_body,
        out_type=jax.ShapeDtypeStruct((NCORE, NP, H), jnp.float32),
        mesh=plsc.VectorSubcoreMesh(core_axis_name="c", subcore_axis_name="s"),
        scratch_types=[
            pltpu.VMEM((K, CHUNK), jnp.int32),
            pltpu.VMEM((K, CHUNK), jnp.int32),
            pltpu.VMEM((NBUF, CHUNK, H), jnp.bfloat16),
            pltpu.VMEM((NBUF, CHUNK, H), jnp.float32),
            pltpu.VMEM_SHARED((NP, H), jnp.float32),
            pltpu.SemaphoreType.DMA((NBUF,)),
            pltpu.SemaphoreType.DMA((NBUF,)),
        ],
        compiler_params=pltpu.CompilerParams(use_tc_tiling_on_sc=False),
    )


def _sc_gather_scatter(proj_flat, fidx_p, dst_p):
    return _make_sc_gather_scatter()(proj_flat, fidx_p, dst_p)


def kernel(h, edge_index, etypes, W_fc, W_attn, W_rel0, W_loop0, b0,
           W_rel1, W_loop1, b1, W_rel2, W_loop2, b2, W_dense, b_dense):
    src = edge_index[0]
    dst = edge_index[1]

    fidx = etypes * NP + src
    pad = EP - E
    fidx_p = jnp.concatenate([fidx, jnp.zeros((pad,), jnp.int32)]).reshape(IDXROWS, CHUNK)
    dst_p = jnp.concatenate([dst, jnp.full((pad,), N, jnp.int32)]).reshape(IDXROWS, CHUNK)
    h_p = jnp.pad(h, ((0, NP - N), (0, 0)))

    proj0, loop0 = _tc_layer0(h_p, W_fc, W_rel0, W_loop0)
    agg0 = _sc_gather_scatter(proj0.reshape(R * NP, H), fidx_p, dst_p)
    proj1, loop1 = _tc_layer_mid(agg0, loop0, b0, W_rel1, W_loop1)
    agg1 = _sc_gather_scatter(proj1.reshape(R * NP, H), fidx_p, dst_p)
    proj2, loop2 = _tc_layer_mid(agg1, loop1, b1, W_rel2, W_loop2)
    agg2 = _sc_gather_scatter(proj2.reshape(R * NP, H), fidx_p, dst_p)
    return _tc_final(agg2, loop2, b2, W_dense, b_dense)


# speedup vs baseline: 1.1758x; 1.0099x over previous
import functools

import jax
import jax.numpy as jnp
from jax import lax
from jax.experimental import pallas as pl
from jax.experimental.pallas import tpu as pltpu
from jax.experimental.pallas import tpu_sc as plsc

N = 10000
E = 320000
R = 8
D_IN = 128
H = 64

NP = 10240
TILE = 640
GRID = NP // TILE

NCORE = 2
NSUB = 16
NWORK = NCORE * NSUB
CHUNK = 128
K = 80
EP = NWORK * K * CHUNK
IDXROWS = EP // CHUNK
ROWS_PER_SUB = NP // NSUB
NBUF = 4


def _layer0_body(h_ref, wfc_ref, wrel_ref, wloop_ref, proj_ref, loop_ref):
    x = lax.dot_general(h_ref[...], wfc_ref[...], (((1,), (1,)), ((), ())),
                        preferred_element_type=jnp.float32)
    loop_ref[...] = jnp.dot(x, wloop_ref[...], preferred_element_type=jnp.float32)
    for r in range(R):
        proj_ref[r] = jnp.dot(x, wrel_ref[r],
                              preferred_element_type=jnp.float32).astype(jnp.bfloat16)


def _layer_mid_body(agg_ref, loopp_ref, b_ref, wrel_ref, wloop_ref,
                    proj_ref, loop_ref):
    x = agg_ref[0] + agg_ref[1] + loopp_ref[...] + b_ref[...]
    x = jnp.maximum(x, 0.0)
    loop_ref[...] = jnp.dot(x, wloop_ref[...], preferred_element_type=jnp.float32)
    for r in range(R):
        proj_ref[r] = jnp.dot(x, wrel_ref[r],
                              preferred_element_type=jnp.float32).astype(jnp.bfloat16)


def _final_body(agg_ref, loopp_ref, b_ref, wd_ref, bd_ref, out_ref, acc_ref):
    i = pl.program_id(0)
    x = agg_ref[0] + agg_ref[1] + loopp_ref[...] + b_ref[...]
    x = jnp.maximum(x, 0.0)
    rows = lax.broadcasted_iota(jnp.int32, (TILE, H), 0) + i * TILE
    x = jnp.where(rows < N, x, 0.0)
    part = jnp.sum(x * wd_ref[...])

    @pl.when(i == 0)
    def _():
        acc_ref[0] = 0.0

    acc_ref[0] += part

    @pl.when(i == GRID - 1)
    def _():
        out_ref[0, 0, 0] = acc_ref[0] + bd_ref[0, 0]


def _tc_layer0(h_p, W_fc, W_rel, W_loop):
    return pl.pallas_call(
        _layer0_body,
        grid=(GRID,),
        in_specs=[
            pl.BlockSpec((TILE, D_IN), lambda i: (i, 0)),
            pl.BlockSpec((H, D_IN), lambda i: (0, 0)),
            pl.BlockSpec((R, H, H), lambda i: (0, 0, 0)),
            pl.BlockSpec((H, H), lambda i: (0, 0)),
        ],
        out_specs=[
            pl.BlockSpec((R, TILE, H), lambda i: (0, i, 0)),
            pl.BlockSpec((TILE, H), lambda i: (i, 0)),
        ],
        out_shape=[
            jax.ShapeDtypeStruct((R, NP, H), jnp.bfloat16),
            jax.ShapeDtypeStruct((NP, H), jnp.float32),
        ],
    )(h_p, W_fc, W_rel, W_loop)


def _tc_layer_mid(agg, loop_prev, b_prev, W_rel, W_loop):
    return pl.pallas_call(
        _layer_mid_body,
        grid=(GRID,),
        in_specs=[
            pl.BlockSpec((2, TILE, H), lambda i: (0, i, 0)),
            pl.BlockSpec((TILE, H), lambda i: (i, 0)),
            pl.BlockSpec((1, H), lambda i: (0, 0)),
            pl.BlockSpec((R, H, H), lambda i: (0, 0, 0)),
            pl.BlockSpec((H, H), lambda i: (0, 0)),
        ],
        out_specs=[
            pl.BlockSpec((R, TILE, H), lambda i: (0, i, 0)),
            pl.BlockSpec((TILE, H), lambda i: (i, 0)),
        ],
        out_shape=[
            jax.ShapeDtypeStruct((R, NP, H), jnp.bfloat16),
            jax.ShapeDtypeStruct((NP, H), jnp.float32),
        ],
    )(agg, loop_prev, b_prev.reshape(1, H), W_rel, W_loop)


def _tc_final(agg, loop_prev, b_prev, W_dense, b_dense):
    return pl.pallas_call(
        _final_body,
        grid=(GRID,),
        in_specs=[
            pl.BlockSpec((2, TILE, H), lambda i: (0, i, 0)),
            pl.BlockSpec((TILE, H), lambda i: (i, 0)),
            pl.BlockSpec((1, H), lambda i: (0, 0)),
            pl.BlockSpec((1, H), lambda i: (0, 0)),
            pl.BlockSpec(memory_space=pltpu.SMEM),
        ],
        out_specs=pl.BlockSpec(memory_space=pltpu.SMEM),
        out_shape=jax.ShapeDtypeStruct((1, 1, 1), jnp.float32),
        scratch_shapes=[pltpu.SMEM((1,), jnp.float32)],
    )(agg, loop_prev, b_prev.reshape(1, H), W_dense, b_dense.reshape(1, 1))


def _sc_body(proj_hbm, fidx_hbm, dst_hbm, out_hbm,
             idx_v, dsti_v, rows_bf, fbuf, agg_sh, gsem, ssem):
    c = lax.axis_index("c")
    s = lax.axis_index("s")
    wid = s * NCORE + c
    row0 = wid * K

    pltpu.sync_copy(fidx_hbm.at[pl.ds(row0, K)], idx_v)
    pltpu.sync_copy(dst_hbm.at[pl.ds(row0, K)], dsti_v)

    zeros16 = jnp.zeros((16,), jnp.float32)

    def _zero_row(i, carry):
        for k in range(H // 16):
            fbuf[0, i, pl.ds(k * 16, 16)] = zeros16
        return carry

    lax.fori_loop(0, CHUNK, _zero_row, 0)
    for k in range(ROWS_PER_SUB // CHUNK):
        pltpu.sync_copy(fbuf.at[0],
                        agg_sh.at[pl.ds(s * ROWS_PER_SUB + k * CHUNK, CHUNK)])
    plsc.subcore_barrier()

    ROWU = 8

    def _widen_chunk(b, carry):
        def _rows(i, carry2):
            base = i * ROWU
            for rr in range(ROWU):
                for k in range(H // 32):
                    v = rows_bf[b, base + rr, pl.ds(k * 32, 32)]
                    fbuf[b, base + rr, pl.ds(k * 32, 32)] = v.astype(jnp.float32)
            return carry2

        return lax.fori_loop(0, CHUNK // ROWU, _rows, carry)

    for b in range(NBUF):
        pltpu.async_copy(proj_hbm.at[idx_v.at[b]], rows_bf.at[b], gsem.at[b])

    def _round(t, carry):
        for b in range(NBUF):
            j = t * NBUF + b
            pltpu.make_async_copy(proj_hbm.at[idx_v.at[j]], rows_bf.at[b],
                                  gsem.at[b]).wait()

            @pl.when(t > 0)
            def _():
                pltpu.make_async_copy(fbuf.at[b], agg_sh.at[dsti_v.at[j]],
                                      ssem.at[b]).wait()

            _widen_chunk(b, 0)
            pltpu.async_copy(fbuf.at[b], agg_sh.at[dsti_v.at[j]], ssem.at[b],
                             add=True)

            @pl.when(t < K // NBUF - 1)
            def _():
                pltpu.async_copy(proj_hbm.at[idx_v.at[(t + 1) * NBUF + b]],
                                 rows_bf.at[b], gsem.at[b])
        return carry

    lax.fori_loop(0, K // NBUF, _round, 0)
    for b in range(NBUF):
        pltpu.make_async_copy(fbuf.at[b], agg_sh.at[dsti_v.at[0]],
                              ssem.at[b]).wait()
    plsc.subcore_barrier()

    pltpu.sync_copy(agg_sh.at[pl.ds(s * ROWS_PER_SUB, ROWS_PER_SUB)],
                    out_hbm.at[c, pl.ds(s * ROWS_PER_SUB, ROWS_PER_SUB)])


@functools.lru_cache(maxsize=None)
def _make_sc_gather_scatter():
    return pl.kernel(
        _sc_body,
        out_type=jax.ShapeDtypeStruct((NCORE, NP, H), jnp.float32),
        mesh=plsc.VectorSubcoreMesh(core_axis_name="c", subcore_axis_name="s"),
        scratch_types=[
            pltpu.VMEM((K, CHUNK), jnp.int32),
            pltpu.VMEM((K, CHUNK), jnp.int32),
            pltpu.VMEM((NBUF, CHUNK, H), jnp.bfloat16),
            pltpu.VMEM((NBUF, CHUNK, H), jnp.float32),
            pltpu.VMEM_SHARED((NP, H), jnp.float32),
            pltpu.SemaphoreType.DMA((NBUF,)),
            pltpu.SemaphoreType.DMA((NBUF,)),
        ],
        compiler_params=pltpu.CompilerParams(use_tc_tiling_on_sc=False),
    )


def _sc_gather_scatter(proj_flat, fidx_p, dst_p):
    return _make_sc_gather_scatter()(proj_flat, fidx_p, dst_p)


def kernel(h, edge_index, etypes, W_fc, W_attn, W_rel0, W_loop0, b0,
           W_rel1, W_loop1, b1, W_rel2, W_loop2, b2, W_dense, b_dense):
    src = edge_index[0]
    dst = edge_index[1]

    fidx = etypes * NP + src
    pad = EP - E
    fidx_p = jnp.concatenate([fidx, jnp.zeros((pad,), jnp.int32)]).reshape(IDXROWS, CHUNK)
    dst_p = jnp.concatenate([dst, jnp.full((pad,), N, jnp.int32)]).reshape(IDXROWS, CHUNK)
    h_p = jnp.pad(h, ((0, NP - N), (0, 0)))

    proj0, loop0 = _tc_layer0(h_p, W_fc, W_rel0, W_loop0)
    agg0 = _sc_gather_scatter(proj0.reshape(R * NP, H), fidx_p, dst_p)
    proj1, loop1 = _tc_layer_mid(agg0, loop0, b0, W_rel1, W_loop1)
    agg1 = _sc_gather_scatter(proj1.reshape(R * NP, H), fidx_p, dst_p)
    proj2, loop2 = _tc_layer_mid(agg1, loop1, b1, W_rel2, W_loop2)
    agg2 = _sc_gather_scatter(proj2.reshape(R * NP, H), fidx_p, dst_p)
    return _tc_final(agg2, loop2, b2, W_dense, b_dense)


# speedup vs baseline: 1.1763x; 1.0004x over previous
import functools

import jax
import jax.numpy as jnp
from jax import lax
from jax.experimental import pallas as pl
from jax.experimental.pallas import tpu as pltpu
from jax.experimental.pallas import tpu_sc as plsc

N = 10000
E = 320000
R = 8
D_IN = 128
H = 64

NP = 10240
TILE = 640
GRID = NP // TILE

NCORE = 2
NSUB = 16
NWORK = NCORE * NSUB
CHUNK = 128
K = 80
EP = NWORK * K * CHUNK
IDXROWS = EP // CHUNK
ROWS_PER_SUB = NP // NSUB
NBUF = 4


def _layer0_body(h_ref, wfc_ref, wrel_ref, wloop_ref, proj_ref, loop_ref):
    x = lax.dot_general(h_ref[...], wfc_ref[...], (((1,), (1,)), ((), ())),
                        preferred_element_type=jnp.float32)
    loop_ref[...] = jnp.dot(x, wloop_ref[...], preferred_element_type=jnp.float32)
    for r in range(R):
        proj_ref[r] = jnp.dot(x, wrel_ref[r],
                              preferred_element_type=jnp.float32).astype(jnp.bfloat16)


def _layer_mid_body(agg_ref, loopp_ref, b_ref, wrel_ref, wloop_ref,
                    proj_ref, loop_ref):
    x = agg_ref[0] + agg_ref[1] + loopp_ref[...] + b_ref[...]
    x = jnp.maximum(x, 0.0)
    loop_ref[...] = jnp.dot(x, wloop_ref[...], preferred_element_type=jnp.float32)
    for r in range(R):
        proj_ref[r] = jnp.dot(x, wrel_ref[r],
                              preferred_element_type=jnp.float32).astype(jnp.bfloat16)


def _final_body(agg_ref, loopp_ref, b_ref, wd_ref, bd_ref, out_ref, acc_ref):
    i = pl.program_id(0)
    x = agg_ref[0] + agg_ref[1] + loopp_ref[...] + b_ref[...]
    x = jnp.maximum(x, 0.0)
    rows = lax.broadcasted_iota(jnp.int32, (TILE, H), 0) + i * TILE
    x = jnp.where(rows < N, x, 0.0)
    part = jnp.sum(x * wd_ref[...])

    @pl.when(i == 0)
    def _():
        acc_ref[0] = 0.0

    acc_ref[0] += part

    @pl.when(i == GRID - 1)
    def _():
        out_ref[0, 0, 0] = acc_ref[0] + bd_ref[0, 0]


def _tc_layer0(h_p, W_fc, W_rel, W_loop):
    return pl.pallas_call(
        _layer0_body,
        grid=(GRID,),
        in_specs=[
            pl.BlockSpec((TILE, D_IN), lambda i: (i, 0)),
            pl.BlockSpec((H, D_IN), lambda i: (0, 0)),
            pl.BlockSpec((R, H, H), lambda i: (0, 0, 0)),
            pl.BlockSpec((H, H), lambda i: (0, 0)),
        ],
        out_specs=[
            pl.BlockSpec((R, TILE, H), lambda i: (0, i, 0)),
            pl.BlockSpec((TILE, H), lambda i: (i, 0)),
        ],
        out_shape=[
            jax.ShapeDtypeStruct((R, NP, H), jnp.bfloat16),
            jax.ShapeDtypeStruct((NP, H), jnp.float32),
        ],
    )(h_p, W_fc, W_rel, W_loop)


def _tc_layer_mid(agg, loop_prev, b_prev, W_rel, W_loop):
    return pl.pallas_call(
        _layer_mid_body,
        grid=(GRID,),
        in_specs=[
            pl.BlockSpec((2, TILE, H), lambda i: (0, i, 0)),
            pl.BlockSpec((TILE, H), lambda i: (i, 0)),
            pl.BlockSpec((1, H), lambda i: (0, 0)),
            pl.BlockSpec((R, H, H), lambda i: (0, 0, 0)),
            pl.BlockSpec((H, H), lambda i: (0, 0)),
        ],
        out_specs=[
            pl.BlockSpec((R, TILE, H), lambda i: (0, i, 0)),
            pl.BlockSpec((TILE, H), lambda i: (i, 0)),
        ],
        out_shape=[
            jax.ShapeDtypeStruct((R, NP, H), jnp.bfloat16),
            jax.ShapeDtypeStruct((NP, H), jnp.float32),
        ],
    )(agg, loop_prev, b_prev.reshape(1, H), W_rel, W_loop)


def _tc_final(agg, loop_prev, b_prev, W_dense, b_dense):
    return pl.pallas_call(
        _final_body,
        grid=(GRID,),
        in_specs=[
            pl.BlockSpec((2, TILE, H), lambda i: (0, i, 0)),
            pl.BlockSpec((TILE, H), lambda i: (i, 0)),
            pl.BlockSpec((1, H), lambda i: (0, 0)),
            pl.BlockSpec((1, H), lambda i: (0, 0)),
            pl.BlockSpec(memory_space=pltpu.SMEM),
        ],
        out_specs=pl.BlockSpec(memory_space=pltpu.SMEM),
        out_shape=jax.ShapeDtypeStruct((1, 1, 1), jnp.float32),
        scratch_shapes=[pltpu.SMEM((1,), jnp.float32)],
    )(agg, loop_prev, b_prev.reshape(1, H), W_dense, b_dense.reshape(1, 1))


def _sc_body(proj_hbm, fidx_hbm, dst_hbm, out_hbm,
             idx_v, dsti_v, rows_bf, fbuf, agg_sh, gsem, ssem):
    c = lax.axis_index("c")
    s = lax.axis_index("s")
    wid = s * NCORE + c
    row0 = wid * K

    pltpu.sync_copy(fidx_hbm.at[pl.ds(row0, K)], idx_v)
    pltpu.sync_copy(dst_hbm.at[pl.ds(row0, K)], dsti_v)

    zeros16 = jnp.zeros((16,), jnp.float32)

    def _zero_row(i, carry):
        for k in range(H // 16):
            fbuf[0, i, pl.ds(k * 16, 16)] = zeros16
        return carry

    lax.fori_loop(0, CHUNK, _zero_row, 0)
    for k in range(ROWS_PER_SUB // CHUNK):
        pltpu.sync_copy(fbuf.at[0],
                        agg_sh.at[pl.ds(s * ROWS_PER_SUB + k * CHUNK, CHUNK)])
    plsc.subcore_barrier()

    ROWU = 8

    def _widen_chunk(b, carry):
        def _rows(i, carry2):
            base = i * ROWU
            for rr in range(ROWU):
                for k in range(H // 32):
                    v = rows_bf[b, base + rr, pl.ds(k * 32, 32)]
                    fbuf[b, base + rr, pl.ds(k * 32, 32)] = v.astype(jnp.float32)
            return carry2

        return lax.fori_loop(0, CHUNK // ROWU, _rows, carry)

    for b in range(NBUF):
        pltpu.async_copy(proj_hbm.at[idx_v.at[b]], rows_bf.at[b], gsem.at[b])

    def _round(t, carry):
        for b in range(NBUF):
            j = t * NBUF + b
            pltpu.make_async_copy(proj_hbm.at[idx_v.at[j]], rows_bf.at[b],
                                  gsem.at[b]).wait()

            @pl.when(t > 0)
            def _():
                pltpu.make_async_copy(fbuf.at[b], agg_sh.at[dsti_v.at[j]],
                                      ssem.at[b]).wait()

            _widen_chunk(b, 0)
            pltpu.async_copy(fbuf.at[b], agg_sh.at[dsti_v.at[j]], ssem.at[b],
                             add=True)

            @pl.when(t < K // NBUF - 1)
            def _():
                pltpu.async_copy(proj_hbm.at[idx_v.at[(t + 1) * NBUF + b]],
                                 rows_bf.at[b], gsem.at[b])
        return carry

    lax.fori_loop(0, K // NBUF, _round, 0)
    for b in range(NBUF):
        pltpu.make_async_copy(fbuf.at[b], agg_sh.at[dsti_v.at[0]],
                              ssem.at[b]).wait()
    plsc.subcore_barrier()

    pltpu.sync_copy(agg_sh.at[pl.ds(s * ROWS_PER_SUB, ROWS_PER_SUB)],
                    out_hbm.at[c, pl.ds(s * ROWS_PER_SUB, ROWS_PER_SUB)])


@functools.lru_cache(maxsize=None)
def _make_sc_gather_scatter():
    return pl.kernel(
        _sc_body,
        out_type=jax.ShapeDtypeStruct((NCORE, NP, H), jnp.float32),
        mesh=plsc.VectorSubcoreMesh(core_axis_name="c", subcore_axis_name="s"),
        scratch_types=[
            pltpu.VMEM((K, CHUNK), jnp.int32),
            pltpu.VMEM((K, CHUNK), jnp.int32),
            pltpu.VMEM((NBUF, CHUNK, H), jnp.bfloat16),
            pltpu.VMEM((NBUF, CHUNK, H), jnp.float32),
            pltpu.VMEM_SHARED((NP, H), jnp.float32),
            pltpu.SemaphoreType.DMA((NBUF,)),
            pltpu.SemaphoreType.DMA((NBUF,)),
        ],
        compiler_params=pltpu.CompilerParams(use_tc_tiling_on_sc=False),
    )


def _sc_gather_scatter(proj_flat, fidx_p, dst_p):
    return _make_sc_gather_scatter()(proj_flat, fidx_p, dst_p)


def kernel(h, edge_index, etypes, W_fc, W_attn, W_rel0, W_loop0, b0,
           W_rel1, W_loop1, b1, W_rel2, W_loop2, b2, W_dense, b_dense):
    src = edge_index[0]
    dst = edge_index[1]

    fidx = etypes * NP + src
    pad = EP - E
    fidx_p = jnp.concatenate([fidx, jnp.zeros((pad,), jnp.int32)]).reshape(IDXROWS, CHUNK)
    pad_dst = N + (jnp.arange(pad, dtype=jnp.int32) % (NP - N))
    dst_p = jnp.concatenate([dst, pad_dst]).reshape(IDXROWS, CHUNK)
    h_p = jnp.pad(h, ((0, NP - N), (0, 0)))

    proj0, loop0 = _tc_layer0(h_p, W_fc, W_rel0, W_loop0)
    agg0 = _sc_gather_scatter(proj0.reshape(R * NP, H), fidx_p, dst_p)
    proj1, loop1 = _tc_layer_mid(agg0, loop0, b0, W_rel1, W_loop1)
    agg1 = _sc_gather_scatter(proj1.reshape(R * NP, H), fidx_p, dst_p)
    proj2, loop2 = _tc_layer_mid(agg1, loop1, b1, W_rel2, W_loop2)
    agg2 = _sc_gather_scatter(proj2.reshape(R * NP, H), fidx_p, dst_p)
    return _tc_final(agg2, loop2, b2, W_dense, b_dense)


# speedup vs baseline: 1.2413x; 1.0553x over previous
import functools

import jax
import jax.numpy as jnp
from jax import lax
from jax.experimental import pallas as pl
from jax.experimental.pallas import tpu as pltpu
from jax.experimental.pallas import tpu_sc as plsc

N = 10000
E = 320000
R = 8
D_IN = 128
H = 64

NP = 10240
TILE = 640
GRID = NP // TILE

NCORE = 2
NSUB = 16
CHUNK = 128
K0 = 92
K1 = 68
IDXROWS = NSUB * (K0 + K1)
IDXROWS_ALLOC = IDXROWS + (K0 - K1)
ROWS_PER_SUB = NP // NSUB
NBUF = 4


def _layer0_body(h_ref, wfc_ref, wrel_ref, wloop_ref, proj_ref, loop_ref):
    x = lax.dot_general(h_ref[...], wfc_ref[...], (((1,), (1,)), ((), ())),
                        preferred_element_type=jnp.float32)
    loop_ref[...] = jnp.dot(x, wloop_ref[...], preferred_element_type=jnp.float32)
    for r in range(R):
        proj_ref[r] = jnp.dot(x, wrel_ref[r],
                              preferred_element_type=jnp.float32).astype(jnp.bfloat16)


def _layer_mid_body(agg_ref, loopp_ref, b_ref, wrel_ref, wloop_ref,
                    proj_ref, loop_ref):
    x = agg_ref[0] + agg_ref[1] + loopp_ref[...] + b_ref[...]
    x = jnp.maximum(x, 0.0)
    loop_ref[...] = jnp.dot(x, wloop_ref[...], preferred_element_type=jnp.float32)
    for r in range(R):
        proj_ref[r] = jnp.dot(x, wrel_ref[r],
                              preferred_element_type=jnp.float32).astype(jnp.bfloat16)


def _final_body(agg_ref, loopp_ref, b_ref, wd_ref, bd_ref, out_ref, acc_ref):
    i = pl.program_id(0)
    x = agg_ref[0] + agg_ref[1] + loopp_ref[...] + b_ref[...]
    x = jnp.maximum(x, 0.0)
    rows = lax.broadcasted_iota(jnp.int32, (TILE, H), 0) + i * TILE
    x = jnp.where(rows < N, x, 0.0)
    part = jnp.sum(x * wd_ref[...])

    @pl.when(i == 0)
    def _():
        acc_ref[0] = 0.0

    acc_ref[0] += part

    @pl.when(i == GRID - 1)
    def _():
        out_ref[0, 0, 0] = acc_ref[0] + bd_ref[0, 0]


def _tc_layer0(h_p, W_fc, W_rel, W_loop):
    return pl.pallas_call(
        _layer0_body,
        grid=(GRID,),
        in_specs=[
            pl.BlockSpec((TILE, D_IN), lambda i: (i, 0)),
            pl.BlockSpec((H, D_IN), lambda i: (0, 0)),
            pl.BlockSpec((R, H, H), lambda i: (0, 0, 0)),
            pl.BlockSpec((H, H), lambda i: (0, 0)),
        ],
        out_specs=[
            pl.BlockSpec((R, TILE, H), lambda i: (0, i, 0)),
            pl.BlockSpec((TILE, H), lambda i: (i, 0)),
        ],
        out_shape=[
            jax.ShapeDtypeStruct((R, NP, H), jnp.bfloat16),
            jax.ShapeDtypeStruct((NP, H), jnp.float32),
        ],
    )(h_p, W_fc, W_rel, W_loop)


def _tc_layer_mid(agg, loop_prev, b_prev, W_rel, W_loop):
    return pl.pallas_call(
        _layer_mid_body,
        grid=(GRID,),
        in_specs=[
            pl.BlockSpec((2, TILE, H), lambda i: (0, i, 0)),
            pl.BlockSpec((TILE, H), lambda i: (i, 0)),
            pl.BlockSpec((1, H), lambda i: (0, 0)),
            pl.BlockSpec((R, H, H), lambda i: (0, 0, 0)),
            pl.BlockSpec((H, H), lambda i: (0, 0)),
        ],
        out_specs=[
            pl.BlockSpec((R, TILE, H), lambda i: (0, i, 0)),
            pl.BlockSpec((TILE, H), lambda i: (i, 0)),
        ],
        out_shape=[
            jax.ShapeDtypeStruct((R, NP, H), jnp.bfloat16),
            jax.ShapeDtypeStruct((NP, H), jnp.float32),
        ],
    )(agg, loop_prev, b_prev.reshape(1, H), W_rel, W_loop)


def _tc_final(agg, loop_prev, b_prev, W_dense, b_dense):
    return pl.pallas_call(
        _final_body,
        grid=(GRID,),
        in_specs=[
            pl.BlockSpec((2, TILE, H), lambda i: (0, i, 0)),
            pl.BlockSpec((TILE, H), lambda i: (i, 0)),
            pl.BlockSpec((1, H), lambda i: (0, 0)),
            pl.BlockSpec((1, H), lambda i: (0, 0)),
            pl.BlockSpec(memory_space=pltpu.SMEM),
        ],
        out_specs=pl.BlockSpec(memory_space=pltpu.SMEM),
        out_shape=jax.ShapeDtypeStruct((1, 1, 1), jnp.float32),
        scratch_shapes=[pltpu.SMEM((1,), jnp.float32)],
    )(agg, loop_prev, b_prev.reshape(1, H), W_dense, b_dense.reshape(1, 1))


def _sc_body(proj_hbm, fidx_hbm, dst_hbm, out_hbm,
             idx_v, dsti_v, rows_bf, fbuf, agg_sh, gsem, ssem):
    c = lax.axis_index("c")
    s = lax.axis_index("s")
    row0 = jnp.where(c == 0, s * K0, NSUB * K0 + s * K1)
    krows = jnp.where(c == 0, K0, K1)
    rounds = krows // NBUF

    pltpu.sync_copy(fidx_hbm.at[pl.ds(row0, K0)], idx_v)
    pltpu.sync_copy(dst_hbm.at[pl.ds(row0, K0)], dsti_v)

    zeros16 = jnp.zeros((16,), jnp.float32)

    def _zero_row(i, carry):
        for k in range(H // 16):
            fbuf[0, i, pl.ds(k * 16, 16)] = zeros16
        return carry

    lax.fori_loop(0, CHUNK, _zero_row, 0)
    for k in range(ROWS_PER_SUB // CHUNK):
        pltpu.sync_copy(fbuf.at[0],
                        agg_sh.at[pl.ds(s * ROWS_PER_SUB + k * CHUNK, CHUNK)])
    plsc.subcore_barrier()

    ROWU = 8

    def _widen_chunk(b, carry):
        def _rows(i, carry2):
            base = i * ROWU
            for rr in range(ROWU):
                for k in range(H // 32):
                    v = rows_bf[b, base + rr, pl.ds(k * 32, 32)]
                    fbuf[b, base + rr, pl.ds(k * 32, 32)] = v.astype(jnp.float32)
            return carry2

        return lax.fori_loop(0, CHUNK // ROWU, _rows, carry)

    for b in range(NBUF):
        pltpu.async_copy(proj_hbm.at[idx_v.at[b]], rows_bf.at[b], gsem.at[b])

    def _round(t, carry):
        for b in range(NBUF):
            j = t * NBUF + b
            pltpu.make_async_copy(proj_hbm.at[idx_v.at[j]], rows_bf.at[b],
                                  gsem.at[b]).wait()

            @pl.when(t > 0)
            def _():
                pltpu.make_async_copy(fbuf.at[b], agg_sh.at[dsti_v.at[j]],
                                      ssem.at[b]).wait()

            _widen_chunk(b, 0)
            pltpu.async_copy(fbuf.at[b], agg_sh.at[dsti_v.at[j]], ssem.at[b],
                             add=True)

            @pl.when(t < rounds - 1)
            def _():
                pltpu.async_copy(proj_hbm.at[idx_v.at[(t + 1) * NBUF + b]],
                                 rows_bf.at[b], gsem.at[b])
        return carry

    lax.fori_loop(0, rounds, _round, 0)
    for b in range(NBUF):
        pltpu.make_async_copy(fbuf.at[b], agg_sh.at[dsti_v.at[krows - NBUF + b]],
                              ssem.at[b]).wait()
    plsc.subcore_barrier()

    pltpu.sync_copy(agg_sh.at[pl.ds(s * ROWS_PER_SUB, ROWS_PER_SUB)],
                    out_hbm.at[c, pl.ds(s * ROWS_PER_SUB, ROWS_PER_SUB)])


@functools.lru_cache(maxsize=None)
def _make_sc_gather_scatter():
    return pl.kernel(
        _sc_body,
        out_type=jax.ShapeDtypeStruct((NCORE, NP, H), jnp.float32),
        mesh=plsc.VectorSubcoreMesh(core_axis_name="c", subcore_axis_name="s"),
        scratch_types=[
            pltpu.VMEM((K0, CHUNK), jnp.int32),
            pltpu.VMEM((K0, CHUNK), jnp.int32),
            pltpu.VMEM((NBUF, CHUNK, H), jnp.bfloat16),
            pltpu.VMEM((NBUF, CHUNK, H), jnp.float32),
            pltpu.VMEM_SHARED((NP, H), jnp.float32),
            pltpu.SemaphoreType.DMA((NBUF,)),
            pltpu.SemaphoreType.DMA((NBUF,)),
        ],
        compiler_params=pltpu.CompilerParams(use_tc_tiling_on_sc=False),
    )


def _sc_gather_scatter(proj_flat, fidx_p, dst_p):
    return _make_sc_gather_scatter()(proj_flat, fidx_p, dst_p)


def kernel(h, edge_index, etypes, W_fc, W_attn, W_rel0, W_loop0, b0,
           W_rel1, W_loop1, b1, W_rel2, W_loop2, b2, W_dense, b_dense):
    src = edge_index[0]
    dst = edge_index[1]

    fidx = etypes * NP + src
    pad = IDXROWS_ALLOC * CHUNK - E
    fidx_p = jnp.concatenate([fidx, jnp.zeros((pad,), jnp.int32)]).reshape(
        IDXROWS_ALLOC, CHUNK)
    pad_dst = N + (jnp.arange(pad, dtype=jnp.int32) % (NP - N))
    dst_p = jnp.concatenate([dst, pad_dst]).reshape(IDXROWS_ALLOC, CHUNK)
    h_p = jnp.pad(h, ((0, NP - N), (0, 0)))

    proj0, loop0 = _tc_layer0(h_p, W_fc, W_rel0, W_loop0)
    agg0 = _sc_gather_scatter(proj0.reshape(R * NP, H), fidx_p, dst_p)
    proj1, loop1 = _tc_layer_mid(agg0, loop0, b0, W_rel1, W_loop1)
    agg1 = _sc_gather_scatter(proj1.reshape(R * NP, H), fidx_p, dst_p)
    proj2, loop2 = _tc_layer_mid(agg1, loop1, b1, W_rel2, W_loop2)
    agg2 = _sc_gather_scatter(proj2.reshape(R * NP, H), fidx_p, dst_p)
    return _tc_final(agg2, loop2, b2, W_dense, b_dense)
